# baseline (speedup 1.0000x reference)
import functools

import jax
import jax.numpy as jnp
from jax import lax
from jax.experimental import pallas as pl
from jax.experimental.pallas import tpu as pltpu
from jax.experimental.pallas import tpu_sc as plsc

B = 256
H = 768
G = 8192
GS = 16
L = G * GS
D = 300
DP = 304
K = 10
KP = 16

GB = 1024
NG = G // GB

NC = 2
NS = 16
NW = NC * NS
RPW = B // NW
NCH = DP // 16


def _tc_body(hidden_ref, w0_ref, b0_ref, w1_ref, b1_ref,
             scores_ref, idx_ref, emb_ref, acc_ref):
    j = pl.program_id(0)
    h = hidden_ref[...]
    blk = jnp.dot(h, w0_ref[...], preferred_element_type=jnp.float32)
    blk = blk + b0_ref[...]
    acc_ref[:, pl.ds(j * GB, GB)] = blk

    @pl.when(j == 0)
    def _():
        emb_ref[...] = (jnp.dot(h, w1_ref[...],
                                preferred_element_type=jnp.float32)
                        + b1_ref[...])

    @pl.when(j == NG - 1)
    def _():
        iota = lax.broadcasted_iota(jnp.int32, (B, G), 1)
        l16 = lax.broadcasted_iota(jnp.int32, (1, KP), 1)

        def body(k, carry):
            sc, ix = carry
            x = acc_ref[...]
            m = jnp.max(x, axis=1, keepdims=True)
            am = jnp.min(jnp.where(x >= m, iota, G), axis=1,
                         keepdims=True)
            acc_ref[...] = jnp.where(iota == am, -jnp.inf, x)
            sc = jnp.where(l16 == k, m, sc)
            ix = jnp.where(l16 == k, am, ix)
            return sc, ix

        sc, ix = lax.fori_loop(
            0, K, body,
            (jnp.zeros((B, KP), jnp.float32), jnp.zeros((B, KP), jnp.int32)))
        scores_ref[...] = sc
        idx_ref[...] = ix


def _tc_stage(hidden, W0, b0_2d, W1p, b1p):
    return pl.pallas_call(
        _tc_body,
        grid=(NG,),
        in_specs=[
            pl.BlockSpec((B, H), lambda j: (0, 0)),
            pl.BlockSpec((H, GB), lambda j: (0, j)),
            pl.BlockSpec((1, GB), lambda j: (0, j)),
            pl.BlockSpec((H, DP), lambda j: (0, 0)),
            pl.BlockSpec((1, DP), lambda j: (0, 0)),
        ],
        out_specs=[
            pl.BlockSpec((B, KP), lambda j: (0, 0)),
            pl.BlockSpec((B, KP), lambda j: (0, 0)),
            pl.BlockSpec((B, DP), lambda j: (0, 0)),
        ],
        out_shape=[
            jax.ShapeDtypeStruct((B, KP), jnp.float32),
            jax.ShapeDtypeStruct((B, KP), jnp.int32),
            jax.ShapeDtypeStruct((B, DP), jnp.float32),
        ],
        scratch_shapes=[pltpu.VMEM((B, G), jnp.float32)],
        compiler_params=pltpu.CompilerParams(
            dimension_semantics=("arbitrary",)),
    )(hidden, W0, b0_2d, W1p, b1p)


def _sigmoid(x):
    return 1.0 / (1.0 + jnp.exp(-x))


def _sc_body(idx_hbm, scores_hbm, emb_hbm, gy_hbm, embed_hbm, out_hbm,
             idx_v, scores_v, emb_v, cand_v, cidx_v, rows_v, out_row, sem):
    wid = lax.axis_index("s") * NC + lax.axis_index("c")
    base = wid * RPW
    iota16 = lax.iota(jnp.int32, 16)

    def row_body(r, _):
        b = base + r
        pltpu.sync_copy(idx_hbm.at[b], idx_v)
        pltpu.sync_copy(scores_hbm.at[b], scores_v)
        pltpu.sync_copy(emb_hbm.at[b], emb_v)
        pltpu.async_copy(gy_hbm.at[idx_v], cand_v, sem).wait()

        ev = [emb_v[pl.ds(16 * j, 16)] for j in range(NCH)]

        def k_body(kk, _):
            cidx_v[...] = cand_v[kk, :]
            pltpu.async_copy(embed_hbm.at[cidx_v], rows_v, sem).wait()
            dots = jnp.zeros((16,), jnp.float32)
            for c in range(GS):
                acc = ev[0] * rows_v[c, pl.ds(0, 16)]
                for j in range(1, NCH):
                    acc = acc + ev[j] * rows_v[c, pl.ds(16 * j, 16)]
                dc = jnp.sum(acc, axis=0)
                dots = dots + jnp.where(iota16 == c, dc, 0.0)
            gsc = plsc.load_gather(scores_v, [jnp.full((16,), kk, jnp.int32)])
            out_chunk = _sigmoid(dots) * _sigmoid(gsc)
            out_row[pl.ds(kk * GS, 16)] = out_chunk
            return 0

        lax.fori_loop(0, K, k_body, 0)
        pltpu.sync_copy(out_row, out_hbm.at[b])
        return 0

    lax.fori_loop(0, RPW, row_body, 0)


def _sc_stage(idx, scores, emb, group_y, embed):
    mesh = plsc.VectorSubcoreMesh(core_axis_name="c", subcore_axis_name="s")
    kern = pl.kernel(
        _sc_body,
        out_type=jax.ShapeDtypeStruct((B, K * GS), jnp.float32),
        mesh=mesh,
        scratch_types=[
            pltpu.VMEM((16,), jnp.int32),
            pltpu.VMEM((16,), jnp.float32),
            pltpu.VMEM((DP,), jnp.float32),
            pltpu.VMEM((16, GS), jnp.int32),
            pltpu.VMEM((16,), jnp.int32),
            pltpu.VMEM((16, DP), jnp.float32),
            pltpu.VMEM((K * GS,), jnp.float32),
            pltpu.SemaphoreType.DMA,
        ],
        compiler_params=pltpu.CompilerParams(needs_layout_passes=False,
                                             use_tc_tiling_on_sc=False),
    )
    return kern(idx, scores, emb, group_y, embed)


@jax.jit
def kernel(hidden, W0, b0, W1, b1, embed, group_y):
    b0_2d = b0.reshape(1, G)
    W1p = jnp.pad(W1, ((0, 0), (0, DP - D)))
    b1p = jnp.pad(b1, (0, DP - D)).reshape(1, DP)
    embed_p = jnp.pad(embed, ((0, 0), (0, DP - D)))
    scores, idx, emb = _tc_stage(hidden, W0, b0_2d, W1p, b1p)
    return _sc_stage(idx, scores, emb, group_y, embed_p)


# speedup vs baseline: 1.6445x; 1.6445x over previous
import functools

import jax
import jax.numpy as jnp
from jax import lax
from jax.experimental import pallas as pl
from jax.experimental.pallas import tpu as pltpu
from jax.experimental.pallas import tpu_sc as plsc

B = 256
H = 768
G = 8192
GS = 16
L = G * GS
D = 300
DP = 304
K = 10
KP = 16

GB = 1024
NG = G // GB

LB = 4096
NL = L // LB
LT = LB // 128

NC = 2
NS = 16
NW = NC * NS
RPW = B // NW


def _tc_body(hidden_ref, w0_ref, b0_ref, w1_ref, b1_ref,
             scores_ref, idx_ref, emb_ref, acc_ref):
    j = pl.program_id(0)
    h = hidden_ref[...]
    blk = jnp.dot(h, w0_ref[...], preferred_element_type=jnp.float32)
    blk = blk + b0_ref[...]
    acc_ref[:, pl.ds(j * GB, GB)] = blk

    @pl.when(j == 0)
    def _():
        emb_ref[...] = (jnp.dot(h, w1_ref[...],
                                preferred_element_type=jnp.float32)
                        + b1_ref[...])

    @pl.when(j == NG - 1)
    def _():
        iota = lax.broadcasted_iota(jnp.int32, (B, G), 1)
        l16 = lax.broadcasted_iota(jnp.int32, (1, KP), 1)

        def body(k, carry):
            sc, ix = carry
            x = acc_ref[...]
            m = jnp.max(x, axis=1, keepdims=True)
            am = jnp.min(jnp.where(x >= m, iota, G), axis=1,
                         keepdims=True)
            acc_ref[...] = jnp.where(iota == am, -jnp.inf, x)
            sc = jnp.where(l16 == k, m, sc)
            ix = jnp.where(l16 == k, am, ix)
            return sc, ix

        sc, ix = lax.fori_loop(
            0, K, body,
            (jnp.zeros((B, KP), jnp.float32), jnp.zeros((B, KP), jnp.int32)))
        scores_ref[...] = sc
        idx_ref[...] = ix


def _tc_stage(hidden, W0, b0_2d, W1p, b1p):
    return pl.pallas_call(
        _tc_body,
        grid=(NG,),
        in_specs=[
            pl.BlockSpec((B, H), lambda j: (0, 0)),
            pl.BlockSpec((H, GB), lambda j: (0, j)),
            pl.BlockSpec((1, GB), lambda j: (0, j)),
            pl.BlockSpec((H, DP), lambda j: (0, 0)),
            pl.BlockSpec((1, DP), lambda j: (0, 0)),
        ],
        out_specs=[
            pl.BlockSpec((B, KP), lambda j: (0, 0)),
            pl.BlockSpec((B, KP), lambda j: (0, 0)),
            pl.BlockSpec((B, DP), lambda j: (0, 0)),
        ],
        out_shape=[
            jax.ShapeDtypeStruct((B, KP), jnp.float32),
            jax.ShapeDtypeStruct((B, KP), jnp.int32),
            jax.ShapeDtypeStruct((B, DP), jnp.float32),
        ],
        scratch_shapes=[pltpu.VMEM((B, G), jnp.float32)],
        compiler_params=pltpu.CompilerParams(
            dimension_semantics=("arbitrary",)),
    )(hidden, W0, b0_2d, W1p, b1p)


def _rescore_body(emb_ref, embed_ref, s_ref):
    e = emb_ref[...][:, :D]
    blk = embed_ref[...]
    s = lax.dot_general(e, blk, (((1,), (1,)), ((), ())),
                        preferred_element_type=jnp.float32)
    for t in range(LT):
        s_ref[:, t, :] = s[:, t * 128:(t + 1) * 128]


def _rescore_stage(emb, embed):
    return pl.pallas_call(
        _rescore_body,
        grid=(NL,),
        in_specs=[
            pl.BlockSpec((B, DP), lambda j: (0, 0)),
            pl.BlockSpec((LB, D), lambda j: (j, 0)),
        ],
        out_specs=pl.BlockSpec((B, LT, 128), lambda j: (0, j, 0)),
        out_shape=jax.ShapeDtypeStruct((B, L // 128, 128), jnp.float32),
        compiler_params=pltpu.CompilerParams(
            dimension_semantics=("arbitrary",)),
    )(emb, embed)


def _sc_body(idx_hbm, scores_hbm, gy_hbm, s_hbm, out_hbm,
             idx_v, scores_v, cand_v, ridx_v, sval_v, out_v, sem):
    wid = lax.axis_index("s") * NC + lax.axis_index("c")
    base = wid * RPW
    iota16 = lax.iota(jnp.int32, 16)

    pltpu.sync_copy(idx_hbm.at[pl.ds(base, RPW)], idx_v)
    pltpu.sync_copy(scores_hbm.at[pl.ds(base, RPW)], scores_v)

    def row_body(r, _):
        b = base + r
        pltpu.async_copy(gy_hbm.at[idx_v.at[r]], cand_v, sem).wait()

        boff = b * (L // 16)
        cvecs = []
        for kk in range(K):
            cv = plsc.load_gather(
                cand_v, [jnp.full((16,), kk, jnp.int32), iota16])
            cvecs.append(cv)
            ridx_v[kk, :] = lax.shift_right_logical(cv, 4) + boff
        copies = [
            pltpu.async_copy(s_hbm.at[ridx_v.at[kk]], sval_v.at[kk], sem)
            for kk in range(K)
        ]
        for c in copies:
            c.wait()

        for kk in range(K):
            sv = plsc.load_gather(
                sval_v, [jnp.full((16,), kk, jnp.int32), iota16,
                         jnp.bitwise_and(cvecs[kk], 15)])
            gsc = plsc.load_gather(
                scores_v, [jnp.full((16,), r, jnp.int32),
                           jnp.full((16,), kk, jnp.int32)])
            chunk = (1.0 / (1.0 + jnp.exp(-sv))) * (1.0 / (1.0 + jnp.exp(-gsc)))
            out_v[r, pl.ds(kk * GS, 16)] = chunk
        return 0

    lax.fori_loop(0, RPW, row_body, 0)
    pltpu.sync_copy(out_v, out_hbm.at[pl.ds(base, RPW)])


def _sc_stage(idx, scores, group_y, s_flat):
    mesh = plsc.VectorSubcoreMesh(core_axis_name="c", subcore_axis_name="s")
    kern = pl.kernel(
        _sc_body,
        out_type=jax.ShapeDtypeStruct((B, K * GS), jnp.float32),
        mesh=mesh,
        scratch_types=[
            pltpu.VMEM((RPW, KP), jnp.int32),
            pltpu.VMEM((RPW, KP), jnp.float32),
            pltpu.VMEM((16, GS), jnp.int32),
            pltpu.VMEM((K, 16), jnp.int32),
            pltpu.VMEM((K, 16, 16), jnp.float32),
            pltpu.VMEM((RPW, K * GS), jnp.float32),
            pltpu.SemaphoreType.DMA,
        ],
        compiler_params=pltpu.CompilerParams(needs_layout_passes=False,
                                             use_tc_tiling_on_sc=False),
    )
    return kern(idx, scores, group_y, s_flat)


@jax.jit
def kernel(hidden, W0, b0, W1, b1, embed, group_y):
    b0_2d = b0.reshape(1, G)
    W1p = jnp.pad(W1, ((0, 0), (0, DP - D)))
    b1p = jnp.pad(b1, (0, DP - D)).reshape(1, DP)
    scores, idx, emb = _tc_stage(hidden, W0, b0_2d, W1p, b1p)
    s3 = _rescore_stage(emb, embed)
    s2 = s3.reshape(B * L // 16, 16)
    return _sc_stage(idx, scores, group_y, s2)


# speedup vs baseline: 1.6470x; 1.0015x over previous
import functools

import jax
import jax.numpy as jnp
from jax import lax
from jax.experimental import pallas as pl
from jax.experimental.pallas import tpu as pltpu
from jax.experimental.pallas import tpu_sc as plsc

B = 256
H = 768
G = 8192
GS = 16
L = G * GS
D = 300
DP = 304
K = 10
KP = 16

GB = 1024
NG = G // GB

LB = 4096
NL = L // LB
LT = LB // 128

NC = 2
NS = 16
NW = NC * NS
RPW = B // NW


def _tc_body(hidden_ref, w0_ref, b0_ref, w1_ref, b1_ref,
             scores_ref, idx_ref, emb_ref, acc_ref):
    j = pl.program_id(0)
    h = hidden_ref[...]
    blk = jnp.dot(h, w0_ref[...], preferred_element_type=jnp.float32)
    blk = blk + b0_ref[...]
    acc_ref[:, pl.ds(j * GB, GB)] = blk

    @pl.when(j == 0)
    def _():
        emb_ref[...] = (jnp.dot(h, w1_ref[...],
                                preferred_element_type=jnp.float32)
                        + b1_ref[...])

    @pl.when(j == NG - 1)
    def _():
        iota = lax.broadcasted_iota(jnp.int32, (B, G), 1)
        l16 = lax.broadcasted_iota(jnp.int32, (1, KP), 1)

        def body(k, carry):
            sc, ix = carry
            x = acc_ref[...]
            m = jnp.max(x, axis=1, keepdims=True)
            am = jnp.min(jnp.where(x >= m, iota, G), axis=1,
                         keepdims=True)
            acc_ref[...] = jnp.where(iota == am, -jnp.inf, x)
            sc = jnp.where(l16 == k, m, sc)
            ix = jnp.where(l16 == k, am, ix)
            return sc, ix

        sc, ix = lax.fori_loop(
            0, K, body,
            (jnp.zeros((B, KP), jnp.float32), jnp.zeros((B, KP), jnp.int32)))
        scores_ref[...] = sc
        idx_ref[...] = ix


def _tc_stage(hidden, W0, b0_2d, W1p, b1p):
    return pl.pallas_call(
        _tc_body,
        grid=(NG,),
        in_specs=[
            pl.BlockSpec((B, H), lambda j: (0, 0)),
            pl.BlockSpec((H, GB), lambda j: (0, j)),
            pl.BlockSpec((1, GB), lambda j: (0, j)),
            pl.BlockSpec((H, DP), lambda j: (0, 0)),
            pl.BlockSpec((1, DP), lambda j: (0, 0)),
        ],
        out_specs=[
            pl.BlockSpec((B, KP), lambda j: (0, 0)),
            pl.BlockSpec((B, KP), lambda j: (0, 0)),
            pl.BlockSpec((B, DP), lambda j: (0, 0)),
        ],
        out_shape=[
            jax.ShapeDtypeStruct((B, KP), jnp.float32),
            jax.ShapeDtypeStruct((B, KP), jnp.int32),
            jax.ShapeDtypeStruct((B, DP), jnp.float32),
        ],
        scratch_shapes=[pltpu.VMEM((B, G), jnp.float32)],
        compiler_params=pltpu.CompilerParams(
            dimension_semantics=("arbitrary",)),
    )(hidden, W0, b0_2d, W1p, b1p)


def _rescore_body(emb_ref, embed_ref, s_ref):
    e = emb_ref[...][:, :D].astype(jnp.bfloat16)
    blk = embed_ref[...].astype(jnp.bfloat16)
    s = lax.dot_general(e, blk, (((1,), (1,)), ((), ())),
                        preferred_element_type=jnp.float32)
    for t in range(LT):
        s_ref[:, t, :] = s[:, t * 128:(t + 1) * 128]


def _rescore_stage(emb, embed):
    return pl.pallas_call(
        _rescore_body,
        grid=(NL,),
        in_specs=[
            pl.BlockSpec((B, DP), lambda j: (0, 0)),
            pl.BlockSpec((LB, D), lambda j: (j, 0)),
        ],
        out_specs=pl.BlockSpec((B, LT, 128), lambda j: (0, j, 0)),
        out_shape=jax.ShapeDtypeStruct((B, L // 128, 128), jnp.float32),
        compiler_params=pltpu.CompilerParams(
            dimension_semantics=("arbitrary",)),
    )(emb, embed)


def _sc_body(idx_hbm, scores_hbm, gy_hbm, s_hbm, out_hbm,
             idx_v, scores_v, cand_v, ridx_v, sval_v, out_v, sem):
    wid = lax.axis_index("s") * NC + lax.axis_index("c")
    base = wid * RPW
    iota16 = lax.iota(jnp.int32, 16)

    pltpu.sync_copy(idx_hbm.at[pl.ds(base, RPW)], idx_v)
    pltpu.sync_copy(scores_hbm.at[pl.ds(base, RPW)], scores_v)

    def row_body(r, _):
        b = base + r
        pltpu.async_copy(gy_hbm.at[idx_v.at[r]], cand_v, sem).wait()

        boff = b * (L // 16)
        cvecs = []
        for kk in range(K):
            cv = plsc.load_gather(
                cand_v, [jnp.full((16,), kk, jnp.int32), iota16])
            cvecs.append(cv)
            ridx_v[kk, :] = lax.shift_right_logical(cv, 4) + boff
        copies = [
            pltpu.async_copy(s_hbm.at[ridx_v.at[kk]], sval_v.at[kk], sem)
            for kk in range(K)
        ]
        for c in copies:
            c.wait()

        for kk in range(K):
            sv = plsc.load_gather(
                sval_v, [jnp.full((16,), kk, jnp.int32), iota16,
                         jnp.bitwise_and(cvecs[kk], 15)])
            gsc = plsc.load_gather(
                scores_v, [jnp.full((16,), r, jnp.int32),
                           jnp.full((16,), kk, jnp.int32)])
            chunk = (1.0 / (1.0 + jnp.exp(-sv))) * (1.0 / (1.0 + jnp.exp(-gsc)))
            out_v[r, pl.ds(kk * GS, 16)] = chunk
        return 0

    lax.fori_loop(0, RPW, row_body, 0)
    pltpu.sync_copy(out_v, out_hbm.at[pl.ds(base, RPW)])


def _sc_stage(idx, scores, group_y, s_flat):
    mesh = plsc.VectorSubcoreMesh(core_axis_name="c", subcore_axis_name="s")
    kern = pl.kernel(
        _sc_body,
        out_type=jax.ShapeDtypeStruct((B, K * GS), jnp.float32),
        mesh=mesh,
        scratch_types=[
            pltpu.VMEM((RPW, KP), jnp.int32),
            pltpu.VMEM((RPW, KP), jnp.float32),
            pltpu.VMEM((16, GS), jnp.int32),
            pltpu.VMEM((K, 16), jnp.int32),
            pltpu.VMEM((K, 16, 16), jnp.float32),
            pltpu.VMEM((RPW, K * GS), jnp.float32),
            pltpu.SemaphoreType.DMA,
        ],
        compiler_params=pltpu.CompilerParams(needs_layout_passes=False,
                                             use_tc_tiling_on_sc=False),
    )
    return kern(idx, scores, group_y, s_flat)


@jax.jit
def kernel(hidden, W0, b0, W1, b1, embed, group_y):
    b0_2d = b0.reshape(1, G)
    W1p = jnp.pad(W1, ((0, 0), (0, DP - D)))
    b1p = jnp.pad(b1, (0, DP - D)).reshape(1, DP)
    scores, idx, emb = _tc_stage(hidden, W0, b0_2d, W1p, b1p)
    s3 = _rescore_stage(emb, embed)
    s2 = s3.reshape(B * L // 16, 16)
    return _sc_stage(idx, scores, group_y, s2)
